# Initial kernel scaffold; baseline (speedup 1.0000x reference)
import jax
import jax.numpy as jnp
from jax.experimental import pallas as pl


def kernel(x, s1_w, s1_sc, s1_sh, s2a_w, s2a_sc, s2a_sh, s2b_w, s2b_sc, s2b_sh, s3_wa, s3_wb, s3_sc, s3_sh):
    raise NotImplementedError("write your pallas kernel here")


# speedup vs baseline: 1.9459x; 1.9459x over previous
import jax
import jax.numpy as jnp
from jax.experimental import pallas as pl
from jax.experimental.pallas import tpu as pltpu

LANES = 128
VMEM_LIMIT = 48 * 1024 * 1024


def _fused_stem(a1p, w1, sc1, sh1, w2a, sc2a, sh2a, w2b, sc2b, sh2b,
                w3a, w3b, sc3, sh3, *, ho2, wo2):
    n = a1p.shape[0]
    m2 = ho2 * wo2
    m1 = 4 * m2
    k1 = a1p.shape[-1]
    hs = ho2 + 1
    ws = wo2 + 1
    cout = 32

    def body(a_ref, w1_ref, sc1_ref, sh1_ref, w2a_ref, sc2a_ref, sh2a_ref,
             w2b_ref, sc2b_ref, sh2b_ref, w3a_ref, w3b_ref, sc3_ref, sh3_ref,
             o_ref, acc_ref, s1_ref, s2a_ref, sd_ref):
        a = a_ref[0].reshape(m1, k1)
        acc_ref[...] = jnp.dot(a, w1_ref[...],
                               preferred_element_type=jnp.float32)
        s1_ref[...] = jnp.maximum(
            acc_ref[...] * sc1_ref[...] + sh1_ref[...], 0.0
        ).astype(jnp.bfloat16)

        s2p = jnp.maximum(
            jnp.maximum(s1_ref[0 * m2:1 * m2], s1_ref[1 * m2:2 * m2]),
            jnp.maximum(s1_ref[2 * m2:3 * m2], s1_ref[3 * m2:4 * m2]))

        s2a_ref[...] = jnp.maximum(
            jnp.dot(s1_ref[...], w2a_ref[...],
                    preferred_element_type=jnp.float32)
            * sc2a_ref[...] + sh2a_ref[...], 0.0).astype(jnp.bfloat16)

        zrow = jnp.zeros((1, ws, 4 * LANES), jnp.bfloat16)
        zcol = jnp.zeros((hs, 1, 4 * LANES), jnp.bfloat16)
        sd_ref[0:1, :, :] = zrow
        sd_ref[hs - 1:hs, :, :] = zrow
        sd_ref[:, 0:1, :] = zcol
        sd_ref[:, ws - 1:ws, :] = zcol
        p00 = s2a_ref[0 * m2:1 * m2].reshape(ho2, wo2, LANES)
        p01 = s2a_ref[1 * m2:2 * m2].reshape(ho2, wo2, LANES)
        p10 = s2a_ref[2 * m2:3 * m2].reshape(ho2, wo2, LANES)
        p11 = s2a_ref[3 * m2:4 * m2].reshape(ho2, wo2, LANES)
        sd_ref[1:hs, 1:ws, 0 * LANES:1 * LANES] = p11
        sd_ref[1:hs, 0:wo2, 1 * LANES:2 * LANES] = p10
        sd_ref[0:ho2, 1:ws, 2 * LANES:3 * LANES] = p01
        sd_ref[0:ho2, 0:wo2, 3 * LANES:4 * LANES] = p00

        first = True
        for dh in (0, 1):
            for dw in (0, 1):
                ag = sd_ref[dh:dh + ho2, dw:dw + wo2, :].reshape(m2, 4 * LANES)
                part = jnp.dot(ag, w2b_ref[2 * dh + dw],
                               preferred_element_type=jnp.float32)
                if first:
                    acc_ref[0:m2] = part
                    first = False
                else:
                    acc_ref[0:m2] += part
        s2b = jnp.maximum(
            acc_ref[0:m2] * sc2b_ref[...] + sh2b_ref[...], 0.0
        ).astype(jnp.bfloat16)

        out = jnp.dot(s2b, w3a_ref[...], preferred_element_type=jnp.float32)
        out = out + jnp.dot(s2p, w3b_ref[...],
                            preferred_element_type=jnp.float32)
        out = jnp.maximum(out * sc3_ref[...] + sh3_ref[...], 0.0)
        o_ref[0] = out[:, :cout]

    return pl.pallas_call(
        body,
        out_shape=jax.ShapeDtypeStruct((n, m2, cout), jnp.float32),
        grid=(n,),
        in_specs=[
            pl.BlockSpec((1, 4, m2, k1), lambda i: (i, 0, 0, 0)),
            pl.BlockSpec((k1, LANES), lambda i: (0, 0)),
            pl.BlockSpec((1, LANES), lambda i: (0, 0)),
            pl.BlockSpec((1, LANES), lambda i: (0, 0)),
            pl.BlockSpec((LANES, LANES), lambda i: (0, 0)),
            pl.BlockSpec((1, LANES), lambda i: (0, 0)),
            pl.BlockSpec((1, LANES), lambda i: (0, 0)),
            pl.BlockSpec((4, 4 * LANES, LANES), lambda i: (0, 0, 0)),
            pl.BlockSpec((1, LANES), lambda i: (0, 0)),
            pl.BlockSpec((1, LANES), lambda i: (0, 0)),
            pl.BlockSpec((LANES, LANES), lambda i: (0, 0)),
            pl.BlockSpec((LANES, LANES), lambda i: (0, 0)),
            pl.BlockSpec((1, LANES), lambda i: (0, 0)),
            pl.BlockSpec((1, LANES), lambda i: (0, 0)),
        ],
        out_specs=pl.BlockSpec((1, m2, cout), lambda i: (i, 0, 0)),
        scratch_shapes=[
            pltpu.VMEM((4 * m2, LANES), jnp.float32),
            pltpu.VMEM((4 * m2, LANES), jnp.bfloat16),
            pltpu.VMEM((4 * m2, LANES), jnp.bfloat16),
            pltpu.VMEM((hs, ws, 4 * LANES), jnp.bfloat16),
        ],
        compiler_params=pltpu.CompilerParams(
            dimension_semantics=("parallel",),
            vmem_limit_bytes=VMEM_LIMIT),
    )(a1p, w1, sc1, sh1, w2a, sc2a, sh2a, w2b, sc2b, sh2b,
      w3a, w3b, sc3, sh3)


def kernel(x, s1_w, s1_sc, s1_sh, s2a_w, s2a_sc, s2a_sh,
           s2b_w, s2b_sc, s2b_sh, s3_wa, s3_wb, s3_sc, s3_sh):
    n, cin, h, w = x.shape
    ho, wo = h // 2, w // 2
    ho2, wo2 = ho // 2, wo // 2

    x_nhwc = jnp.transpose(x, (0, 2, 3, 1)).astype(jnp.bfloat16)
    xp = jnp.pad(x_nhwc, ((0, 0), (1, 1), (1, 1), (0, 0)))
    s2d = jnp.concatenate(
        [xp[:, ph::2, pw::2, :] for ph in (0, 1) for pw in (0, 1)], axis=-1)
    a1 = jnp.concatenate(
        [s2d[:, dh:dh + ho, dw:dw + wo, :] for dh in (0, 1) for dw in (0, 1)],
        axis=-1)
    a1p = jnp.stack(
        [a1[:, ph::2, pw::2, :] for ph in (0, 1) for pw in (0, 1)], axis=1)
    a1p = a1p.reshape(n, 4, ho2 * wo2, a1p.shape[-1])

    w1_48 = s1_w.reshape(4 * s1_w.shape[1], s1_w.shape[2])

    out_rows = _fused_stem(
        a1p, w1_48, s1_sc, s1_sh, s2a_w, s2a_sc, s2a_sh,
        s2b_w, s2b_sc, s2b_sh, s3_wa, s3_wb, s3_sc, s3_sh,
        ho2=ho2, wo2=wo2)

    out = out_rows.reshape(n, ho2, wo2, out_rows.shape[-1])
    return jnp.transpose(out, (0, 3, 1, 2))
